# Initial kernel scaffold; baseline (speedup 1.0000x reference)
import jax
import jax.numpy as jnp
from jax import lax
from jax.experimental import pallas as pl
from jax.experimental.pallas import tpu as pltpu
from jax.experimental.pallas import tpu_sc as plsc

NC = 2
NS = 16
NW = NC * NS

B_ROWS = 16384
B_COLS = 50
B_TOTAL = B_ROWS * B_COLS
D = 32

G = 128
BPW = B_TOTAL // NW
NG = BPW // G
K = 8
NR = NG // K


def _gather_body(idx_hbm, table_hbm, out_hbm, idx_v, rows_v, sem):
    wid = lax.axis_index("s") * NC + lax.axis_index("c")
    gbase = wid * NG

    pltpu.sync_copy(idx_hbm.at[pl.ds(gbase, NG)], idx_v)

    def round_body(r, carry):
        copies = []
        for j in range(K):
            cp = pltpu.async_copy(
                table_hbm.at[idx_v.at[r * K + j]],
                rows_v.at[pl.ds(j * G, G)],
                sem,
            )
            copies.append(cp)
        for cp in copies:
            cp.wait()
        row0 = (gbase + r * K) * G
        pltpu.sync_copy(rows_v, out_hbm.at[pl.ds(row0, K * G)])
        return carry

    lax.fori_loop(0, NR, round_body, 0)


@jax.jit
def _embed(idx2d, weights):
    fn = pl.kernel(
        _gather_body,
        out_type=jax.ShapeDtypeStruct((B_TOTAL, D), jnp.float32),
        mesh=plsc.VectorSubcoreMesh(core_axis_name="c", subcore_axis_name="s"),
        scratch_types=[
            pltpu.VMEM((NG, G), jnp.int32),
            pltpu.VMEM((K * G, D), jnp.float32),
            pltpu.SemaphoreType.DMA,
        ],
    )
    return fn(idx2d, weights)


def kernel(inputs, index, weights):
    idx2d = inputs.reshape(B_TOTAL // G, G)
    out = _embed(idx2d, weights)
    return out.reshape(B_ROWS, B_COLS, D)


# speedup vs baseline: 1.1031x; 1.1031x over previous
import jax
import jax.numpy as jnp
from jax import lax
from jax.experimental import pallas as pl
from jax.experimental.pallas import tpu as pltpu
from jax.experimental.pallas import tpu_sc as plsc

NC = 2
NS = 16
NW = NC * NS

B_ROWS = 16384
B_COLS = 50
B_TOTAL = B_ROWS * B_COLS
D = 32

G = 128
BPW = B_TOTAL // NW
NG = BPW // G
K = 8
NR = NG // K


def _gather_body(idx_hbm, table_hbm, out_hbm, idx_v, rows_v, sem):
    wid = lax.axis_index("s") * NC + lax.axis_index("c")
    gbase = wid * NG

    pltpu.sync_copy(idx_hbm.at[pl.ds(gbase, NG)], idx_v)

    def round_body(r, carry):
        copies = []
        for j in range(K):
            cp = pltpu.async_copy(
                table_hbm.at[idx_v.at[r * K + j]],
                rows_v.at[pl.ds(j * G, G)],
                sem,
            )
            copies.append(cp)
        for cp in copies:
            cp.wait()
        row0 = (gbase + r * K) * G
        pltpu.sync_copy(rows_v, out_hbm.at[pl.ds(row0, K * G)])
        return carry

    lax.fori_loop(0, NR, round_body, 0)


@jax.jit
def _embed(idx2d, weights):
    fn = pl.kernel(
        _gather_body,
        out_type=jax.ShapeDtypeStruct((B_TOTAL, D), jnp.float32),
        mesh=plsc.VectorSubcoreMesh(core_axis_name="c", subcore_axis_name="s"),
        compiler_params=pltpu.CompilerParams(use_tc_tiling_on_sc=False),
        scratch_types=[
            pltpu.VMEM((NG, G), jnp.int32),
            pltpu.VMEM((K * G, D), jnp.float32),
            pltpu.SemaphoreType.DMA,
        ],
    )
    return fn(idx2d, weights)


def kernel(inputs, index, weights):
    idx2d = inputs.reshape(B_TOTAL // G, G)
    out = _embed(idx2d, weights)
    return out.reshape(B_ROWS, B_COLS, D)


# speedup vs baseline: 1.1099x; 1.0062x over previous
import jax
import jax.numpy as jnp
from jax import lax
from jax.experimental import pallas as pl
from jax.experimental.pallas import tpu as pltpu
from jax.experimental.pallas import tpu_sc as plsc

NC = 2
NS = 16
NW = NC * NS

B_ROWS = 16384
B_COLS = 50
B_TOTAL = B_ROWS * B_COLS
D = 32

G = 128
BPW = B_TOTAL // NW
NG = BPW // G
K = 10
CH = K * G
NR = NG // K
NRH = NR // 2


def _gather_body(idx_hbm, table_hbm, out_hbm, idx_v, rows_v, g0, g1, o0, o1):
    wid = lax.axis_index("s") * NC + lax.axis_index("c")
    gbase = wid * NG

    pltpu.sync_copy(idx_hbm.at[pl.ds(gbase, NG)], idx_v)

    def fire_g(r, p, sem):
        for j in range(K):
            pltpu.async_copy(
                table_hbm.at[idx_v.at[r * K + j]],
                rows_v.at[p, pl.ds(j * G, G)],
                sem,
            )

    def wait_g(p, sem):
        pltpu.make_async_copy(out_hbm.at[pl.ds(0, CH)], rows_v.at[p], sem).wait()

    def fire_o(r, p, sem):
        row0 = (gbase + r * K) * G
        pltpu.async_copy(rows_v.at[p], out_hbm.at[pl.ds(row0, CH)], sem)

    def wait_o(p, sem):
        pltpu.make_async_copy(rows_v.at[p], out_hbm.at[pl.ds(0, CH)], sem).wait()

    fire_g(0, 0, g0)

    def body(i, carry):
        r0 = 2 * i
        r1 = r0 + 1
        wait_g(0, g0)

        @pl.when(i > 0)
        def _():
            wait_o(1, o1)

        fire_g(r1, 1, g1)
        fire_o(r0, 0, o0)
        wait_g(1, g1)
        wait_o(0, o0)

        @pl.when(i < NRH - 1)
        def _():
            fire_g(r0 + 2, 0, g0)

        fire_o(r1, 1, o1)
        return carry

    lax.fori_loop(0, NRH, body, 0)
    wait_o(1, o1)


@jax.jit
def _embed(idx2d, weights):
    fn = pl.kernel(
        _gather_body,
        out_type=jax.ShapeDtypeStruct((B_TOTAL, D), jnp.float32),
        mesh=plsc.VectorSubcoreMesh(core_axis_name="c", subcore_axis_name="s"),
        compiler_params=pltpu.CompilerParams(use_tc_tiling_on_sc=False),
        scratch_types=[
            pltpu.VMEM((NG, G), jnp.int32),
            pltpu.VMEM((2, CH, D), jnp.float32),
            pltpu.SemaphoreType.DMA,
            pltpu.SemaphoreType.DMA,
            pltpu.SemaphoreType.DMA,
            pltpu.SemaphoreType.DMA,
        ],
    )
    return fn(idx2d, weights)


def kernel(inputs, index, weights):
    idx2d = inputs.reshape(B_TOTAL // G, G)
    out = _embed(idx2d, weights)
    return out.reshape(B_ROWS, B_COLS, D)


# speedup vs baseline: 1.4843x; 1.3373x over previous
import jax
import jax.numpy as jnp
from jax import lax
from jax.experimental import pallas as pl
from jax.experimental.pallas import tpu as pltpu
from jax.experimental.pallas import tpu_sc as plsc

NC = 2
NS = 16
NW = NC * NS

B = 16384
S = 50
V = 1_000_000
D = 32

BPW = B // NW
G = 128
NH = BPW // G
NPAIR = S * NH // 2


def _body(idx3, table2, out3, idxq_v, r3_v, rows0, rows1, tbuf0, tbuf1,
          g0, g1, o0, o1):
    w = lax.axis_index("s") * NC + lax.axis_index("c")
    iota = lax.iota(jnp.int32, 16)
    b0 = w * BPW

    pltpu.sync_copy(idx3.at[:, pl.ds(4 * w, 4), :], idxq_v)

    def split(s, carry):
        for j in range(NH):
            for u in range(8):
                x = idxq_v[s, j, pl.ds(16 * u, 16)]
                r3_v[s, j, pl.ds(16 * u, 16)] = (x & 3) * D
                idxq_v[s, j, pl.ds(16 * u, 16)] = x >> 2
        return carry

    lax.fori_loop(0, S, split, 0)

    def fire_g(s, h, rows, sem):
        pltpu.async_copy(table2.at[idxq_v.at[s, h]], rows, sem)

    def wait_g(rows, sem):
        pltpu.make_async_copy(table2.at[pl.ds(0, G)], rows, sem).wait()

    def fire_o(s, h, tbuf, sem):
        pltpu.async_copy(tbuf, out3.at[s, :, pl.ds(b0 + G * h, G)], sem)

    def wait_o(tbuf, sem):
        pltpu.make_async_copy(tbuf, out3.at[0, :, pl.ds(0, G)], sem).wait()

    def extract(s, h, rows, tbuf):
        def vbody(v, carry):
            rv = 16 * v + iota
            base = r3_v[s, h, pl.ds(16 * v, 16)]
            for d in range(D):
                x = plsc.load_gather(rows, [rv, base + d])
                tbuf[d, pl.ds(16 * v, 16)] = x
            return carry

        lax.fori_loop(0, G // 16, vbody, 0)

    fire_g(0, 0, rows0, g0)

    def pair(i, carry):
        s = i // 2
        h0 = 2 * (i % 2)
        h1 = h0 + 1
        wait_g(rows0, g0)
        fire_g(s, h1, rows1, g1)

        @pl.when(i > 0)
        def _():
            wait_o(tbuf0, o0)

        extract(s, h0, rows0, tbuf0)
        fire_o(s, h0, tbuf0, o0)
        wait_g(rows1, g1)

        @pl.when(i < NPAIR - 1)
        def _():
            s_n = (i + 1) // 2
            h_n = 2 * ((i + 1) % 2)
            fire_g(s_n, h_n, rows0, g0)

        @pl.when(i > 0)
        def _():
            wait_o(tbuf1, o1)

        extract(s, h1, rows1, tbuf1)
        fire_o(s, h1, tbuf1, o1)
        return carry

    lax.fori_loop(0, NPAIR, pair, 0)
    wait_o(tbuf0, o0)
    wait_o(tbuf1, o1)


def kernel(inputs, index, weights):
    table2 = weights.reshape(V // 4, 128)
    idx3 = inputs.T.reshape(S, B // 128, 128)

    p = pl.kernel(
        _body,
        out_type=jax.ShapeDtypeStruct((S, D, B), jnp.float32),
        mesh=plsc.VectorSubcoreMesh(core_axis_name="c", subcore_axis_name="s"),
        compiler_params=pltpu.CompilerParams(use_tc_tiling_on_sc=False,
                                             needs_layout_passes=False),
        scratch_types=[
            pltpu.VMEM((S, NH, 128), jnp.int32),
            pltpu.VMEM((S, NH, 128), jnp.int32),
            pltpu.VMEM((G, 128), jnp.float32),
            pltpu.VMEM((G, 128), jnp.float32),
            pltpu.VMEM((D, G), jnp.float32),
            pltpu.VMEM((D, G), jnp.float32),
            pltpu.SemaphoreType.DMA,
            pltpu.SemaphoreType.DMA,
            pltpu.SemaphoreType.DMA,
            pltpu.SemaphoreType.DMA,
        ],
    )
    out3 = p(idx3, table2)
    return out3.transpose(2, 0, 1)
